# Initial kernel scaffold; baseline (speedup 1.0000x reference)
import jax
import jax.numpy as jnp
from jax.experimental import pallas as pl


def kernel(pred_similarities, kernel_mask_ndi_labels):
    raise NotImplementedError("write your pallas kernel here")


# speedup vs baseline: 2.2311x; 2.2311x over previous
import functools
import math

import jax
import jax.numpy as jnp
from jax import lax
from jax.experimental import pallas as pl
from jax.experimental.pallas import tpu as pltpu
from jax.experimental.pallas import tpu_sc as plsc

B, C, H, W = 2, 4, 512, 512
N = H * W
NC, NS, L = 2, 16, 16
NW = NC * NS
PIX = N // NS
NBIN = 5
ACC = (1 + C) * NBIN * L
LOG10 = math.log(10.0)

_mesh = plsc.VectorSubcoreMesh(
    core_axis_name="c", subcore_axis_name="s", num_cores=NC, num_subcores=NS
)


@functools.partial(
    pl.kernel,
    out_type=jax.ShapeDtypeStruct((NW, ACC), jnp.float32),
    mesh=_mesh,
    scratch_types=[
        pltpu.VMEM((PIX,), jnp.int32),
        pltpu.VMEM((C, PIX), jnp.float32),
        pltpu.VMEM((ACC,), jnp.float32),
    ],
    compiler_params=pltpu.CompilerParams(needs_layout_passes=False),
)
def _binned_sums(pred_hbm, lab_hbm, out_hbm, labv, chv, acc):
    cid = lax.axis_index("c")
    sid = lax.axis_index("s")
    batch = cid
    base = sid * PIX

    zeros = jnp.zeros((L,), jnp.float32)
    for j in range(ACC // L):
        acc[pl.ds(j * L, L)] = zeros

    pltpu.sync_copy(lab_hbm.at[batch, pl.ds(base, PIX)], labv)
    for ch in range(C):
        pltpu.sync_copy(pred_hbm.at[batch, ch, pl.ds(base, PIX)], chv.at[ch])

    iota = lax.iota(jnp.int32, L)
    ones = jnp.ones((L,), jnp.float32)

    def body(i, carry):
        lab = labv[pl.ds(i * L, L)]
        idx = lab * L + iota
        plsc.addupdate_scatter(acc, [idx], ones)
        for ch in range(C):
            x = chv[ch, pl.ds(i * L, L)]
            plsc.addupdate_scatter(acc, [idx + (1 + ch) * NBIN * L], x)
        return carry

    lax.fori_loop(0, PIX // L, body, 0)

    wid = cid * NS + sid
    pltpu.sync_copy(acc, out_hbm.at[wid])


def _epilogue(part_ref, out_ref):
    x = part_ref[...]
    total = jnp.float32(0.0)
    for b in range(B):
        rows = x[b * NS:(b + 1) * NS, :]
        cnt = [jnp.sum(rows[:, r * L:(r + 1) * L]) for r in range(NBIN)]
        kmax = jnp.float32(0.0)
        for r in range(1, NBIN):
            kmax = jnp.where(cnt[r] > 0.0, jnp.float32(r), kmax)
        accum = jnp.float32(0.0)
        for r in range(1, NBIN):
            s2 = jnp.float32(0.0)
            for ch in range(C):
                col = (1 + ch) * NBIN * L + r * L
                s = jnp.sum(rows[:, col:col + L])
                s2 = s2 + s * s
            nrm = jnp.sqrt(s2)
            fr = jnp.log(jnp.maximum(3.0 - nrm, 0.0) ** 2 + 1.0)
            valid = (jnp.float32(r) <= kmax).astype(jnp.float32)
            accum = accum + valid * cnt[r] * (fr - LOG10)
        pairs = kmax * (kmax - 1.0) * 0.5
        total = total + pairs * jnp.float32(B * N * LOG10) + (kmax - 1.0) * accum
    out_ref[0, 0] = total


def kernel(pred_similarities, kernel_mask_ndi_labels):
    pred = pred_similarities.reshape(B, C, N)
    lab = kernel_mask_ndi_labels.reshape(B, N).astype(jnp.int32)
    parts = _binned_sums(pred, lab)
    res = pl.pallas_call(
        _epilogue,
        out_shape=jax.ShapeDtypeStruct((1, 1), jnp.float32),
        out_specs=pl.BlockSpec(memory_space=pltpu.SMEM),
    )(parts)
    return res[0, 0]


# speedup vs baseline: 2.9579x; 1.3258x over previous
import functools
import math

import jax
import jax.numpy as jnp
from jax import lax
from jax.experimental import pallas as pl
from jax.experimental.pallas import tpu as pltpu
from jax.experimental.pallas import tpu_sc as plsc

B, C, H, W = 2, 4, 512, 512
N = H * W
NC, NS, L = 2, 16, 16
NW = NC * NS
PIX = N // NS
NBIN = 5
ACC = (1 + C) * NBIN * L
LOG10 = math.log(10.0)

_mesh = plsc.VectorSubcoreMesh(
    core_axis_name="c", subcore_axis_name="s", num_cores=NC, num_subcores=NS
)


@functools.partial(
    pl.kernel,
    out_type=jax.ShapeDtypeStruct((NW, ACC), jnp.float32),
    mesh=_mesh,
    scratch_types=[
        pltpu.VMEM((PIX,), jnp.int32),
        pltpu.VMEM((C, PIX), jnp.float32),
        pltpu.VMEM((NBIN * L,), jnp.float32),
        pltpu.VMEM((NBIN * L,), jnp.float32),
        pltpu.VMEM((NBIN * L,), jnp.float32),
        pltpu.VMEM((NBIN * L,), jnp.float32),
        pltpu.VMEM((NBIN * L,), jnp.float32),
        pltpu.VMEM((ACC,), jnp.float32),
    ],
    compiler_params=pltpu.CompilerParams(needs_layout_passes=False),
)
def _binned_sums(pred_hbm, lab_hbm, out_hbm, labv, chv, cnt, a0, a1, a2, a3, stage):
    cid = lax.axis_index("c")
    sid = lax.axis_index("s")
    batch = cid
    base = sid * PIX
    accs = [cnt, a0, a1, a2, a3]

    zeros = jnp.zeros((L,), jnp.float32)
    for a in accs:
        for j in range(NBIN):
            a[pl.ds(j * L, L)] = zeros

    pltpu.sync_copy(lab_hbm.at[batch, pl.ds(base, PIX)], labv)
    for ch in range(C):
        pltpu.sync_copy(pred_hbm.at[batch, ch, pl.ds(base, PIX)], chv.at[ch])

    iota = lax.iota(jnp.int32, L)
    ones = jnp.ones((L,), jnp.float32)

    @plsc.parallel_loop(0, PIX // L, unroll=8)
    def _(i):
        lab = labv[pl.ds(i * L, L)]
        idx = lab * L + iota
        plsc.addupdate_scatter(cnt, [idx], ones)
        for ch in range(C):
            x = chv[ch, pl.ds(i * L, L)]
            plsc.addupdate_scatter(accs[1 + ch], [idx], x)

    wid = cid * NS + sid
    for j, a in enumerate(accs):
        for k in range(NBIN):
            stage[pl.ds((j * NBIN + k) * L, L)] = a[pl.ds(k * L, L)]
    pltpu.sync_copy(stage, out_hbm.at[wid])


def _epilogue(part_ref, out_ref):
    x = part_ref[...]
    total = jnp.float32(0.0)
    for b in range(B):
        rows = x[b * NS:(b + 1) * NS, :]
        cnt = [jnp.sum(rows[:, r * L:(r + 1) * L]) for r in range(NBIN)]
        kmax = jnp.float32(0.0)
        for r in range(1, NBIN):
            kmax = jnp.where(cnt[r] > 0.0, jnp.float32(r), kmax)
        accum = jnp.float32(0.0)
        for r in range(1, NBIN):
            s2 = jnp.float32(0.0)
            for ch in range(C):
                col = (1 + ch) * NBIN * L + r * L
                s = jnp.sum(rows[:, col:col + L])
                s2 = s2 + s * s
            nrm = jnp.sqrt(s2)
            fr = jnp.log(jnp.maximum(3.0 - nrm, 0.0) ** 2 + 1.0)
            valid = (jnp.float32(r) <= kmax).astype(jnp.float32)
            accum = accum + valid * cnt[r] * (fr - LOG10)
        pairs = kmax * (kmax - 1.0) * 0.5
        total = total + pairs * jnp.float32(B * N * LOG10) + (kmax - 1.0) * accum
    out_ref[0, 0] = total


def kernel(pred_similarities, kernel_mask_ndi_labels):
    pred = pred_similarities.reshape(B, C, N)
    lab = kernel_mask_ndi_labels.reshape(B, N).astype(jnp.int32)
    parts = _binned_sums(pred, lab)
    res = pl.pallas_call(
        _epilogue,
        out_shape=jax.ShapeDtypeStruct((1, 1), jnp.float32),
        out_specs=pl.BlockSpec(memory_space=pltpu.SMEM),
    )(parts)
    return res[0, 0]


# speedup vs baseline: 4.2995x; 1.4536x over previous
import functools
import math

import jax
import jax.numpy as jnp
from jax import lax
from jax.experimental import pallas as pl
from jax.experimental.pallas import tpu as pltpu
from jax.experimental.pallas import tpu_sc as plsc

B, C, H, W = 2, 4, 512, 512
N = H * W
NC, NS, L = 2, 16, 16
NW = NC * NS
PIX = N // NS
NBIN = 5
ACC = (1 + C) * NBIN * L
LOG10 = math.log(10.0)

_mesh = plsc.VectorSubcoreMesh(
    core_axis_name="c", subcore_axis_name="s", num_cores=NC, num_subcores=NS
)


@functools.partial(
    pl.kernel,
    out_type=jax.ShapeDtypeStruct((NW, ACC), jnp.float32),
    mesh=_mesh,
    scratch_types=[
        pltpu.VMEM((H // NS, W), jnp.int32),
        pltpu.VMEM((C, H // NS, W), jnp.float32),
        pltpu.VMEM((NBIN * L,), jnp.float32),
        pltpu.VMEM((NBIN * L,), jnp.float32),
        pltpu.VMEM((NBIN * L,), jnp.float32),
        pltpu.VMEM((NBIN * L,), jnp.float32),
        pltpu.VMEM((NBIN * L,), jnp.float32),
        pltpu.VMEM((ACC,), jnp.float32),
    ],
    compiler_params=pltpu.CompilerParams(needs_layout_passes=False),
)
def _binned_sums(pred_hbm, lab_hbm, out_hbm, labv, chv, cnt, a0, a1, a2, a3, stage):
    cid = lax.axis_index("c")
    sid = lax.axis_index("s")
    batch = cid
    rows = H // NS
    r0 = sid * rows
    accs = [cnt, a0, a1, a2, a3]

    zeros = jnp.zeros((L,), jnp.float32)
    for a in accs:
        for j in range(NBIN):
            a[pl.ds(j * L, L)] = zeros

    pltpu.sync_copy(lab_hbm.at[batch, 0, pl.ds(r0, rows), :], labv)
    for ch in range(C):
        pltpu.sync_copy(pred_hbm.at[batch, ch, pl.ds(r0, rows), :], chv.at[ch])

    iota = lax.iota(jnp.int32, L)
    ones = jnp.ones((L,), jnp.float32)
    vecs_per_row = W // L

    @plsc.parallel_loop(0, PIX // L, unroll=8)
    def _(i):
        r = i // vecs_per_row
        col = (i % vecs_per_row) * L
        lab = labv[r, pl.ds(col, L)]
        idx = lab * L + iota
        plsc.addupdate_scatter(cnt, [idx], ones)
        for ch in range(C):
            x = chv[ch, r, pl.ds(col, L)]
            plsc.addupdate_scatter(accs[1 + ch], [idx], x)

    wid = cid * NS + sid
    for j, a in enumerate(accs):
        for k in range(NBIN):
            stage[pl.ds((j * NBIN + k) * L, L)] = a[pl.ds(k * L, L)]
    pltpu.sync_copy(stage, out_hbm.at[wid])


def _epilogue(part_ref, out_ref):
    x = part_ref[...]
    total = jnp.float32(0.0)
    for b in range(B):
        rows = x[b * NS:(b + 1) * NS, :]
        cnt = [jnp.sum(rows[:, r * L:(r + 1) * L]) for r in range(NBIN)]
        kmax = jnp.float32(0.0)
        for r in range(1, NBIN):
            kmax = jnp.where(cnt[r] > 0.0, jnp.float32(r), kmax)
        accum = jnp.float32(0.0)
        for r in range(1, NBIN):
            s2 = jnp.float32(0.0)
            for ch in range(C):
                col = (1 + ch) * NBIN * L + r * L
                s = jnp.sum(rows[:, col:col + L])
                s2 = s2 + s * s
            nrm = jnp.sqrt(s2)
            fr = jnp.log(jnp.maximum(3.0 - nrm, 0.0) ** 2 + 1.0)
            valid = (jnp.float32(r) <= kmax).astype(jnp.float32)
            accum = accum + valid * cnt[r] * (fr - LOG10)
        pairs = kmax * (kmax - 1.0) * 0.5
        total = total + pairs * jnp.float32(B * N * LOG10) + (kmax - 1.0) * accum
    out_ref[0, 0] = total


def kernel(pred_similarities, kernel_mask_ndi_labels):
    parts = _binned_sums(pred_similarities, kernel_mask_ndi_labels)
    res = pl.pallas_call(
        _epilogue,
        out_shape=jax.ShapeDtypeStruct((1, 1), jnp.float32),
        out_specs=pl.BlockSpec(memory_space=pltpu.SMEM),
    )(parts)
    return res[0, 0]


# speedup vs baseline: 4.7232x; 1.0985x over previous
import functools
import math

import jax
import jax.numpy as jnp
from jax import lax
from jax.experimental import pallas as pl
from jax.experimental.pallas import tpu as pltpu
from jax.experimental.pallas import tpu_sc as plsc

B, C, H, W = 2, 4, 512, 512
N = H * W
NC, NS, L = 2, 16, 16
NW = NC * NS
PIX = N // NS
NBIN = 5
ACC = (1 + C) * NBIN * L
LOG10 = math.log(10.0)

_mesh = plsc.VectorSubcoreMesh(
    core_axis_name="c", subcore_axis_name="s", num_cores=NC, num_subcores=NS
)


@functools.partial(
    pl.kernel,
    out_type=jax.ShapeDtypeStruct((NW, ACC), jnp.float32),
    mesh=_mesh,
    scratch_types=[
        pltpu.VMEM((2, H // NS // 4, W), jnp.int32),
        pltpu.VMEM((2, C, H // NS // 4, W), jnp.float32),
        pltpu.VMEM((NBIN * L,), jnp.float32),
        pltpu.VMEM((NBIN * L,), jnp.float32),
        pltpu.VMEM((NBIN * L,), jnp.float32),
        pltpu.VMEM((NBIN * L,), jnp.float32),
        pltpu.VMEM((NBIN * L,), jnp.float32),
        pltpu.VMEM((ACC,), jnp.float32),
        pltpu.SemaphoreType.DMA,
        pltpu.SemaphoreType.DMA,
    ],
    compiler_params=pltpu.CompilerParams(needs_layout_passes=False),
)
def _binned_sums(pred_hbm, lab_hbm, out_hbm, labv, chv, cnt, a0, a1, a2, a3,
                 stage, sem0, sem1):
    cid = lax.axis_index("c")
    sid = lax.axis_index("s")
    batch = cid
    rows = H // NS
    NCHK = 4
    CR = rows // NCHK
    r0 = sid * rows
    accs = [cnt, a0, a1, a2, a3]
    sems = [sem0, sem1]

    zeros = jnp.zeros((L,), jnp.float32)
    for a in accs:
        for j in range(NBIN):
            a[pl.ds(j * L, L)] = zeros

    iota = lax.iota(jnp.int32, L)
    ones = jnp.ones((L,), jnp.float32)
    vecs_per_row = W // L

    def issue(g):
        buf = g & 1
        rb = r0 + g * CR
        hl = pltpu.async_copy(lab_hbm.at[batch, 0, pl.ds(rb, CR), :],
                              labv.at[buf], sems[buf])
        hp = pltpu.async_copy(pred_hbm.at[batch, :, pl.ds(rb, CR), :],
                              chv.at[buf], sems[buf])
        return (hl, hp)

    handles = issue(0)
    for g in range(NCHK):
        nxt = issue(g + 1) if g + 1 < NCHK else None
        for h in handles:
            h.wait()
        handles = nxt
        buf = g & 1

        @plsc.parallel_loop(0, CR * W // L, unroll=8)
        def _(i):
            r = i // vecs_per_row
            col = (i % vecs_per_row) * L
            lab = labv[buf, r, pl.ds(col, L)]
            idx = lab * L + iota
            plsc.addupdate_scatter(cnt, [idx], ones)
            for ch in range(C):
                x = chv[buf, ch, r, pl.ds(col, L)]
                plsc.addupdate_scatter(accs[1 + ch], [idx], x)

    wid = cid * NS + sid
    for j, a in enumerate(accs):
        for k in range(NBIN):
            stage[pl.ds((j * NBIN + k) * L, L)] = a[pl.ds(k * L, L)]
    pltpu.sync_copy(stage, out_hbm.at[wid])


def _epilogue(part_ref, out_ref):
    x = part_ref[...]
    total = jnp.float32(0.0)
    for b in range(B):
        rows = x[b * NS:(b + 1) * NS, :]
        cnt = [jnp.sum(rows[:, r * L:(r + 1) * L]) for r in range(NBIN)]
        kmax = jnp.float32(0.0)
        for r in range(1, NBIN):
            kmax = jnp.where(cnt[r] > 0.0, jnp.float32(r), kmax)
        accum = jnp.float32(0.0)
        for r in range(1, NBIN):
            s2 = jnp.float32(0.0)
            for ch in range(C):
                col = (1 + ch) * NBIN * L + r * L
                s = jnp.sum(rows[:, col:col + L])
                s2 = s2 + s * s
            nrm = jnp.sqrt(s2)
            fr = jnp.log(jnp.maximum(3.0 - nrm, 0.0) ** 2 + 1.0)
            valid = (jnp.float32(r) <= kmax).astype(jnp.float32)
            accum = accum + valid * cnt[r] * (fr - LOG10)
        pairs = kmax * (kmax - 1.0) * 0.5
        total = total + pairs * jnp.float32(B * N * LOG10) + (kmax - 1.0) * accum
    out_ref[0, 0] = total


def kernel(pred_similarities, kernel_mask_ndi_labels):
    parts = _binned_sums(pred_similarities, kernel_mask_ndi_labels)
    res = pl.pallas_call(
        _epilogue,
        out_shape=jax.ShapeDtypeStruct((1, 1), jnp.float32),
        out_specs=pl.BlockSpec(memory_space=pltpu.SMEM),
    )(parts)
    return res[0, 0]


# speedup vs baseline: 4.7258x; 1.0005x over previous
import functools
import math

import jax
import jax.numpy as jnp
from jax import lax
from jax.experimental import pallas as pl
from jax.experimental.pallas import tpu as pltpu
from jax.experimental.pallas import tpu_sc as plsc

B, C, H, W = 2, 4, 512, 512
N = H * W
NC, NS, L = 2, 16, 16
NW = NC * NS
PIX = N // NS
NBIN = 5
ACC = (1 + C) * NBIN * L
LOG10 = math.log(10.0)

_mesh = plsc.VectorSubcoreMesh(
    core_axis_name="c", subcore_axis_name="s", num_cores=NC, num_subcores=NS
)


@functools.partial(
    pl.kernel,
    out_type=jax.ShapeDtypeStruct((NW, ACC), jnp.float32),
    mesh=_mesh,
    scratch_types=[
        pltpu.VMEM((2, H // NS // 4, W), jnp.int32),
        pltpu.VMEM((2, C, H // NS // 4, W), jnp.float32),
        pltpu.VMEM((NBIN * L,), jnp.float32),
        pltpu.VMEM((NBIN * L,), jnp.float32),
        pltpu.VMEM((NBIN * L,), jnp.float32),
        pltpu.VMEM((NBIN * L,), jnp.float32),
        pltpu.VMEM((NBIN * L,), jnp.float32),
        pltpu.VMEM((ACC,), jnp.float32),
        pltpu.SemaphoreType.DMA,
        pltpu.SemaphoreType.DMA,
    ],
    compiler_params=pltpu.CompilerParams(needs_layout_passes=False),
)
def _binned_sums(pred_hbm, lab_hbm, out_hbm, labv, chv, cnt, a0, a1, a2, a3,
                 stage, sem0, sem1):
    cid = lax.axis_index("c")
    sid = lax.axis_index("s")
    batch = cid
    rows = H // NS
    NCHK = 4
    CR = rows // NCHK
    r0 = sid * rows
    accs = [cnt, a0, a1, a2, a3]
    sems = [sem0, sem1]

    zeros = jnp.zeros((L,), jnp.float32)
    for a in accs:
        for j in range(NBIN):
            a[pl.ds(j * L, L)] = zeros

    iota = lax.iota(jnp.int32, L)
    ones = jnp.ones((L,), jnp.float32)
    vecs_per_row = W // L

    def issue(g):
        buf = g & 1
        rb = r0 + g * CR
        hl = pltpu.async_copy(lab_hbm.at[batch, 0, pl.ds(rb, CR), :],
                              labv.at[buf], sems[buf])
        hp = pltpu.async_copy(pred_hbm.at[batch, :, pl.ds(rb, CR), :],
                              chv.at[buf], sems[buf])
        return (hl, hp)

    handles = issue(0)
    cnt_vecs = [zeros] * (NBIN - 1)
    for g in range(NCHK):
        nxt = issue(g + 1) if g + 1 < NCHK else None
        for h in handles:
            h.wait()
        handles = nxt
        buf = g & 1

        @plsc.parallel_loop(0, CR * W // L, unroll=8, carry=cnt_vecs)
        def _(i, cv):
            r = i // vecs_per_row
            col = (i % vecs_per_row) * L
            lab = labv[buf, r, pl.ds(col, L)]
            idx = lab * L + iota
            cv = [c + jnp.where(lab == (j + 1), 1.0, 0.0)
                  for j, c in enumerate(cv)]
            for ch in range(C):
                x = chv[buf, ch, r, pl.ds(col, L)]
                plsc.addupdate_scatter(accs[1 + ch], [idx], x)
            return cv

        cnt_vecs = _

    for j, cv in enumerate(cnt_vecs):
        cnt[pl.ds((j + 1) * L, L)] = cv

    wid = cid * NS + sid
    for j, a in enumerate(accs):
        for k in range(NBIN):
            stage[pl.ds((j * NBIN + k) * L, L)] = a[pl.ds(k * L, L)]
    pltpu.sync_copy(stage, out_hbm.at[wid])


def _epilogue(part_ref, out_ref):
    x = part_ref[...]
    total = jnp.float32(0.0)
    for b in range(B):
        rows = x[b * NS:(b + 1) * NS, :]
        cnt = [jnp.sum(rows[:, r * L:(r + 1) * L]) for r in range(NBIN)]
        kmax = jnp.float32(0.0)
        for r in range(1, NBIN):
            kmax = jnp.where(cnt[r] > 0.0, jnp.float32(r), kmax)
        accum = jnp.float32(0.0)
        for r in range(1, NBIN):
            s2 = jnp.float32(0.0)
            for ch in range(C):
                col = (1 + ch) * NBIN * L + r * L
                s = jnp.sum(rows[:, col:col + L])
                s2 = s2 + s * s
            nrm = jnp.sqrt(s2)
            fr = jnp.log(jnp.maximum(3.0 - nrm, 0.0) ** 2 + 1.0)
            valid = (jnp.float32(r) <= kmax).astype(jnp.float32)
            accum = accum + valid * cnt[r] * (fr - LOG10)
        pairs = kmax * (kmax - 1.0) * 0.5
        total = total + pairs * jnp.float32(B * N * LOG10) + (kmax - 1.0) * accum
    out_ref[0, 0] = total


def kernel(pred_similarities, kernel_mask_ndi_labels):
    parts = _binned_sums(pred_similarities, kernel_mask_ndi_labels)
    res = pl.pallas_call(
        _epilogue,
        out_shape=jax.ShapeDtypeStruct((1, 1), jnp.float32),
        out_specs=pl.BlockSpec(memory_space=pltpu.SMEM),
    )(parts)
    return res[0, 0]
